# baseline (speedup 1.0000x reference)
import functools

import jax
import jax.numpy as jnp
from jax import lax
from jax.experimental import pallas as pl
from jax.experimental.pallas import tpu as pltpu
from jax.experimental.pallas import tpu_sc as plsc

_EPS = 1e-5
_CHUNK = 32
_TC_BLOCK_IMGS = 8
_SC_IMGS = 16
_NW = 32
_SC_ROWS = 128
_SC_DMA_ROWS = 32

_LN2 = 0.6931471805599453
_TWO23 = 8388608.0
_GC = (-1.13380952 * _TWO23, 2.01071821 * _TWO23,
       -1.02948618 * _TWO23, 0.153912422 * _TWO23)


def _rawlog2(x):
    bits = lax.bitcast_convert_type(x, jnp.int32)
    base = bits.astype(jnp.float32)
    m = lax.bitcast_convert_type(
        (bits & 0x7FFFFF) | 0x3F800000, jnp.float32)
    g = jnp.float32(_GC[3])
    g = g * m + _GC[2]
    g = g * m + _GC[1]
    g = g * m + _GC[0]
    return base + g


def _tc_body(p_ref, t_ref, out_ref, acc_ref, *, nsteps):
    i = pl.program_id(0)
    rows = p_ref.shape[1]

    def inner(k, carry):
        s_p, s_pt, s_t, s_f, s_ft = carry
        sl = pl.ds(k * _CHUNK, _CHUNK)
        p = p_ref[:, sl, :]
        tf = t_ref[:, sl, :].astype(jnp.float32)
        la = jnp.log(p + _EPS)
        lb = jnp.log((1.0 + _EPS) - p)
        ent = p * (la - lb) + lb
        s_p = s_p + jnp.sum(p, axis=0)
        s_pt = s_pt + jnp.sum(p * tf, axis=0)
        s_t = s_t + jnp.sum(tf, axis=0)
        s_f = s_f + jnp.sum(ent, axis=0)
        s_ft = s_ft + jnp.sum(ent * tf, axis=0)
        return s_p, s_pt, s_t, s_f, s_ft

    zero = jnp.zeros((_CHUNK, 256), jnp.float32)
    sums = jax.lax.fori_loop(0, rows // _CHUNK, inner,
                             (zero, zero, zero, zero, zero))
    block = jnp.stack(sums, axis=0)

    @pl.when(i == 0)
    def _init():
        acc_ref[...] = block

    @pl.when(i > 0)
    def _accum():
        acc_ref[...] = acc_ref[...] + block

    @pl.when(i == nsteps - 1)
    def _finish():
        acc = acc_ref[...]
        for q in range(5):
            out_ref[0, q] = jnp.sum(acc[q])


def _tc_partial_sums(pred, neg_target, n_imgs):
    nsteps = n_imgs // _TC_BLOCK_IMGS
    return pl.pallas_call(
        functools.partial(_tc_body, nsteps=nsteps),
        grid=(nsteps,),
        in_specs=[
            pl.BlockSpec((_TC_BLOCK_IMGS, 256, 256), lambda i: (i, 0, 0)),
            pl.BlockSpec((_TC_BLOCK_IMGS, 256, 256), lambda i: (i, 0, 0)),
        ],
        out_specs=pl.BlockSpec((1, 5), lambda i: (0, 0),
                               memory_space=pltpu.SMEM),
        out_shape=jax.ShapeDtypeStruct((1, 5), jnp.float32),
        scratch_shapes=[pltpu.VMEM((5, _CHUNK, 256), jnp.float32)],
    )(pred, neg_target)


def _sc_body(p_hbm, t_hbm, out_hbm, pbuf, tbuf, obuf, psem, tsem, *,
             img0, n_imgs):
    c = lax.axis_index("c")
    s = lax.axis_index("s")
    wid = s * 2 + c
    wpi = _NW // n_imgs
    rows_pw = 256 // wpi
    img = img0 + wid // wpi
    row0 = (wid % wpi) * rows_pw

    nstages = rows_pw // _SC_DMA_ROWS

    def start_stage(st, buf_slot):
        sl = pl.ds(row0 + st * _SC_DMA_ROWS, _SC_DMA_ROWS)
        cp_p = pltpu.make_async_copy(
            p_hbm.at[img, sl, :], pbuf.at[buf_slot], psem)
        cp_t = pltpu.make_async_copy(
            t_hbm.at[img, sl, :], tbuf.at[buf_slot], tsem)
        cp_p.start()
        cp_t.start()
        return cp_p, cp_t

    zero = jnp.zeros((16,), jnp.float32)
    accs = (zero, zero, zero, zero, zero)

    cur = start_stage(0, 0)
    for st in range(nstages):
        nxt = start_stage(st + 1, (st + 1) % 2) if st + 1 < nstages else None
        cur[0].wait()
        cur[1].wait()
        slot = st % 2

        def row_body(r, carry):
            s_p, s_pt, s_t, s_f, s_ft = carry
            for cc in range(16):
                colsl = pl.ds(cc * 16, 16)
                p = pbuf[slot, r, colsl]
                tf = tbuf[slot, r, colsl].astype(jnp.float32)
                ra = _rawlog2(p + _EPS)
                rb = _rawlog2((1.0 + _EPS) - p)
                ent = p * (ra - rb) + (rb - 127.0 * _TWO23)
                s_p = s_p + p
                s_pt = s_pt + p * tf
                s_t = s_t + tf
                s_f = s_f + ent
                s_ft = s_ft + ent * tf
            return s_p, s_pt, s_t, s_f, s_ft

        accs = lax.fori_loop(0, _SC_DMA_ROWS, row_body, accs)
        cur = nxt

    for q in range(5):
        obuf[q, :] = accs[q]
    pltpu.sync_copy(obuf, out_hbm.at[wid])


def _sc_partial_sums(pred, neg_target, img0, n_imgs):
    call = pl.kernel(
        functools.partial(_sc_body, img0=img0, n_imgs=n_imgs),
        out_type=jax.ShapeDtypeStruct((_NW, 5, 16), jnp.float32),
        mesh=plsc.VectorSubcoreMesh(core_axis_name="c", subcore_axis_name="s"),
        scratch_types=[
            pltpu.VMEM((2, _SC_DMA_ROWS, 256), jnp.float32),
            pltpu.VMEM((2, _SC_DMA_ROWS, 256), jnp.int32),
            pltpu.VMEM((5, 16), jnp.float32),
            pltpu.SemaphoreType.DMA,
            pltpu.SemaphoreType.DMA,
        ],
    )
    return call(pred, neg_target)


def kernel(pred, neg_target):
    n_total = pred.size
    num_image = pred.shape[0]
    tc = _tc_partial_sums(pred, neg_target, num_image - _SC_IMGS)
    half = _SC_IMGS // 2
    sc_a = _sc_partial_sums(pred, neg_target, num_image - _SC_IMGS, half)
    sc_b = _sc_partial_sums(pred, neg_target, num_image - half, half)
    scs = jnp.sum(sc_a, axis=(0, 2)) + jnp.sum(sc_b, axis=(0, 2))
    sc_scale = _LN2 / _TWO23
    s_p = tc[0, 0] + scs[0]
    s_pt = tc[0, 1] + scs[1]
    s_t = tc[0, 2] + scs[2]
    s_f = tc[0, 3] + scs[3] * sc_scale
    s_ft = tc[0, 4] + scs[4] * sc_scale
    s_pneg = s_p - s_pt
    c_neg = n_total - s_t
    s_fneg = s_f - s_ft
    pos_avg = s_pt / jnp.maximum(s_t, 1.0)
    l1 = jnp.log(pos_avg + _EPS)
    l2 = jnp.log(1.0 - pos_avg + _EPS)
    entropy = -(s_fneg - l1 * s_pneg - l2 * (c_neg - s_pneg))
    return entropy / num_image


# speedup vs baseline: 1.2059x; 1.2059x over previous
import functools

import jax
import jax.numpy as jnp
from jax import lax
from jax.experimental import pallas as pl
from jax.experimental.pallas import tpu as pltpu
from jax.experimental.pallas import tpu_sc as plsc

_EPS = 1e-5
_CHUNK = 32
_TC_BLOCK_IMGS = 4
_SC_IMGS = 4
_NW = 32
_SC_ROWS = 128
_SC_DMA_ROWS = 32

_LN2 = 0.6931471805599453
_TWO23 = 8388608.0
_GC = (-1.13380952 * _TWO23, 2.01071821 * _TWO23,
       -1.02948618 * _TWO23, 0.153912422 * _TWO23)


def _rawlog2(x):
    bits = lax.bitcast_convert_type(x, jnp.int32)
    base = bits.astype(jnp.float32)
    m = lax.bitcast_convert_type(
        (bits & 0x7FFFFF) | 0x3F800000, jnp.float32)
    g = jnp.float32(_GC[3])
    g = g * m + _GC[2]
    g = g * m + _GC[1]
    g = g * m + _GC[0]
    return base + g


def _tc_body(p_ref, t_ref, out_ref, acc_ref, *, nsteps):
    i = pl.program_id(0)
    rows = p_ref.shape[1]

    def inner(k, carry):
        s_p, s_pt, s_t, s_f, s_ft = carry
        sl = pl.ds(k * _CHUNK, _CHUNK)
        p = p_ref[:, sl, :]
        tf = t_ref[:, sl, :].astype(jnp.float32)
        la = jnp.log(p + _EPS)
        lb = jnp.log((1.0 + _EPS) - p)
        ent = p * (la - lb) + lb
        s_p = s_p + jnp.sum(p, axis=0)
        s_pt = s_pt + jnp.sum(p * tf, axis=0)
        s_t = s_t + jnp.sum(tf, axis=0)
        s_f = s_f + jnp.sum(ent, axis=0)
        s_ft = s_ft + jnp.sum(ent * tf, axis=0)
        return s_p, s_pt, s_t, s_f, s_ft

    zero = jnp.zeros((_CHUNK, 256), jnp.float32)
    sums = jax.lax.fori_loop(0, rows // _CHUNK, inner,
                             (zero, zero, zero, zero, zero))
    block = jnp.stack(sums, axis=0)

    @pl.when(i == 0)
    def _init():
        acc_ref[...] = block

    @pl.when(i > 0)
    def _accum():
        acc_ref[...] = acc_ref[...] + block

    @pl.when(i == nsteps - 1)
    def _finish():
        acc = acc_ref[...]
        for q in range(5):
            out_ref[0, q] = jnp.sum(acc[q])


def _tc_partial_sums(pred, neg_target, n_imgs):
    nsteps = n_imgs // _TC_BLOCK_IMGS
    return pl.pallas_call(
        functools.partial(_tc_body, nsteps=nsteps),
        grid=(nsteps,),
        in_specs=[
            pl.BlockSpec((_TC_BLOCK_IMGS, 256, 256), lambda i: (i, 0, 0)),
            pl.BlockSpec((_TC_BLOCK_IMGS, 256, 256), lambda i: (i, 0, 0)),
        ],
        out_specs=pl.BlockSpec((1, 5), lambda i: (0, 0),
                               memory_space=pltpu.SMEM),
        out_shape=jax.ShapeDtypeStruct((1, 5), jnp.float32),
        scratch_shapes=[pltpu.VMEM((5, _CHUNK, 256), jnp.float32)],
    )(pred, neg_target)


def _sc_body(p_hbm, t_hbm, out_hbm, pbuf, tbuf, obuf, psem, tsem, *,
             img0, n_imgs):
    c = lax.axis_index("c")
    s = lax.axis_index("s")
    wid = s * 2 + c
    wpi = _NW // n_imgs
    rows_pw = 256 // wpi
    img = img0 + wid // wpi
    row0 = (wid % wpi) * rows_pw

    nstages = rows_pw // _SC_DMA_ROWS

    def start_stage(st, buf_slot):
        sl = pl.ds(row0 + st * _SC_DMA_ROWS, _SC_DMA_ROWS)
        cp_p = pltpu.make_async_copy(
            p_hbm.at[img, sl, :], pbuf.at[buf_slot], psem)
        cp_t = pltpu.make_async_copy(
            t_hbm.at[img, sl, :], tbuf.at[buf_slot], tsem)
        cp_p.start()
        cp_t.start()
        return cp_p, cp_t

    zero = jnp.zeros((16,), jnp.float32)
    accs = (zero, zero, zero, zero, zero)

    cur = start_stage(0, 0)
    for st in range(nstages):
        nxt = start_stage(st + 1, (st + 1) % 2) if st + 1 < nstages else None
        cur[0].wait()
        cur[1].wait()
        slot = st % 2

        def row_body(r, carry):
            s_p, s_pt, s_t, s_f, s_ft = carry
            for cc in range(16):
                colsl = pl.ds(cc * 16, 16)
                p = pbuf[slot, r, colsl]
                tf = tbuf[slot, r, colsl].astype(jnp.float32)
                ra = _rawlog2(p + _EPS)
                rb = _rawlog2((1.0 + _EPS) - p)
                ent = p * (ra - rb) + (rb - 127.0 * _TWO23)
                s_p = s_p + p
                s_pt = s_pt + p * tf
                s_t = s_t + tf
                s_f = s_f + ent
                s_ft = s_ft + ent * tf
            return s_p, s_pt, s_t, s_f, s_ft

        accs = lax.fori_loop(0, _SC_DMA_ROWS, row_body, accs)
        cur = nxt

    for q in range(5):
        obuf[q, :] = accs[q]
    pltpu.sync_copy(obuf, out_hbm.at[wid])


def _sc_partial_sums(pred, neg_target, img0, n_imgs):
    call = pl.kernel(
        functools.partial(_sc_body, img0=img0, n_imgs=n_imgs),
        out_type=jax.ShapeDtypeStruct((_NW, 5, 16), jnp.float32),
        mesh=plsc.VectorSubcoreMesh(core_axis_name="c", subcore_axis_name="s"),
        scratch_types=[
            pltpu.VMEM((2, _SC_DMA_ROWS, 256), jnp.float32),
            pltpu.VMEM((2, _SC_DMA_ROWS, 256), jnp.int32),
            pltpu.VMEM((5, 16), jnp.float32),
            pltpu.SemaphoreType.DMA,
            pltpu.SemaphoreType.DMA,
        ],
    )
    return call(pred, neg_target)


def kernel(pred, neg_target):
    n_total = pred.size
    num_image = pred.shape[0]
    tc = _tc_partial_sums(pred, neg_target, num_image - _SC_IMGS)
    sc = _sc_partial_sums(pred, neg_target, num_image - _SC_IMGS, _SC_IMGS)
    scs = jnp.sum(sc, axis=(0, 2))
    sc_scale = _LN2 / _TWO23
    s_p = tc[0, 0] + scs[0]
    s_pt = tc[0, 1] + scs[1]
    s_t = tc[0, 2] + scs[2]
    s_f = tc[0, 3] + scs[3] * sc_scale
    s_ft = tc[0, 4] + scs[4] * sc_scale
    s_pneg = s_p - s_pt
    c_neg = n_total - s_t
    s_fneg = s_f - s_ft
    pos_avg = s_pt / jnp.maximum(s_t, 1.0)
    l1 = jnp.log(pos_avg + _EPS)
    l2 = jnp.log(1.0 - pos_avg + _EPS)
    entropy = -(s_fneg - l1 * s_pneg - l2 * (c_neg - s_pneg))
    return entropy / num_image


# speedup vs baseline: 2.8457x; 2.3598x over previous
import functools

import jax
import jax.numpy as jnp
from jax.experimental import pallas as pl
from jax.experimental.pallas import tpu as pltpu

_EPS = 1e-5
_CHUNK = 32


def _loss_body(p_ref, t_ref, out_ref, acc_ref, *, nsteps, n_total, num_image):
    i = pl.program_id(0)
    rows = p_ref.shape[1]

    def inner(k, carry):
        s_p, s_pt, s_t, s_f, s_ft = carry
        sl = pl.ds(k * _CHUNK, _CHUNK)
        p = p_ref[:, sl, :]
        tf = t_ref[:, sl, :].astype(jnp.float32)
        la = jnp.log(p + _EPS)
        lb = jnp.log((1.0 + _EPS) - p)
        ent = p * (la - lb) + lb
        s_p = s_p + jnp.sum(p, axis=0)
        s_pt = s_pt + jnp.sum(p * tf, axis=0)
        s_t = s_t + jnp.sum(tf, axis=0)
        s_f = s_f + jnp.sum(ent, axis=0)
        s_ft = s_ft + jnp.sum(ent * tf, axis=0)
        return s_p, s_pt, s_t, s_f, s_ft

    zero = jnp.zeros((_CHUNK, 256), jnp.float32)
    sums = jax.lax.fori_loop(0, rows // _CHUNK, inner,
                             (zero, zero, zero, zero, zero))
    block = jnp.stack(sums, axis=0)

    @pl.when(i == 0)
    def _init():
        acc_ref[...] = block

    @pl.when(i > 0)
    def _accum():
        acc_ref[...] = acc_ref[...] + block

    @pl.when(i == nsteps - 1)
    def _finish():
        acc = acc_ref[...]
        s_p = jnp.sum(acc[0])
        s_pt = jnp.sum(acc[1])
        s_t = jnp.sum(acc[2])
        s_f = jnp.sum(acc[3])
        s_ft = jnp.sum(acc[4])
        s_pneg = s_p - s_pt
        c_neg = n_total - s_t
        s_fneg = s_f - s_ft
        pos_avg = s_pt / jnp.maximum(s_t, 1.0)
        l1 = jnp.log(pos_avg + _EPS)
        l2 = jnp.log(1.0 - pos_avg + _EPS)
        entropy = -(s_fneg - l1 * s_pneg - l2 * (c_neg - s_pneg))
        out_ref[0, 0] = entropy / num_image


def kernel(pred, neg_target):
    n_total = pred.size
    num_image = pred.shape[0]
    block_imgs = 8
    nsteps = num_image // block_imgs
    out = pl.pallas_call(
        functools.partial(
            _loss_body,
            nsteps=nsteps,
            n_total=float(n_total),
            num_image=float(num_image),
        ),
        grid=(nsteps,),
        in_specs=[
            pl.BlockSpec((block_imgs, 256, 256), lambda i: (i, 0, 0)),
            pl.BlockSpec((block_imgs, 256, 256), lambda i: (i, 0, 0)),
        ],
        out_specs=pl.BlockSpec((1, 1), lambda i: (0, 0), memory_space=pltpu.SMEM),
        out_shape=jax.ShapeDtypeStruct((1, 1), jnp.float32),
        scratch_shapes=[pltpu.VMEM((5, _CHUNK, 256), jnp.float32)],
    )(pred, neg_target)
    return out[0, 0]
